# Initial kernel scaffold; baseline (speedup 1.0000x reference)
import jax
import jax.numpy as jnp
from jax.experimental import pallas as pl


def kernel(x, edge_index, weight, bias):
    raise NotImplementedError("write your pallas kernel here")


# speedup vs baseline: 13.6908x; 13.6908x over previous
import functools

import jax
import jax.numpy as jnp
from jax import lax
from jax.experimental import pallas as pl
from jax.experimental.pallas import tpu as pltpu
from jax.experimental.pallas import tpu_sc as plsc

N = 10000
F = 128
E = 320000

NC = 2
NS = 16
CH = 128

CHUNKS_32 = -(-E // (NC * NS * CH))
E_PAD = NC * NS * CH * CHUNKS_32
CHUNKS_16 = E_PAD // (NS * CH)

ROWS_PER_TILE = -(-(N + 1) // (NS * 16)) * 16
ACC_N = NS * ROWS_PER_TILE

_MESH = plsc.VectorSubcoreMesh(core_axis_name="c", subcore_axis_name="s")


@functools.partial(
    pl.kernel,
    out_type=jax.ShapeDtypeStruct((ACC_N,), jnp.float32),
    mesh=_MESH,
    scratch_types=[
        pltpu.VMEM((CH,), jnp.int32),
        pltpu.VMEM((CH,), jnp.float32),
        pltpu.VMEM((ROWS_PER_TILE,), jnp.float32),
        pltpu.VMEM_SHARED((ACC_N,), jnp.float32),
    ],
)
def _deg_hist(row_hbm, hist_out, idxbuf, ones, dbuf, hist):
    cid = lax.axis_index("c")
    sid = lax.axis_index("s")

    def init(i, c):
        ones[pl.ds(i * 16, 16)] = jnp.ones((16,), jnp.float32)
        return c
    lax.fori_loop(0, CH // 16, init, 0)

    def zero(i, c):
        dbuf[pl.ds(i * 16, 16)] = jnp.zeros((16,), jnp.float32)
        return c
    lax.fori_loop(0, ROWS_PER_TILE // 16, zero, 0)
    pltpu.sync_copy(dbuf, hist.at[pl.ds(sid * ROWS_PER_TILE, ROWS_PER_TILE)])
    plsc.subcore_barrier()

    ebase = sid * (CHUNKS_16 * CH)

    def body(j, c):
        pltpu.sync_copy(row_hbm.at[pl.ds(ebase + j * CH, CH)], idxbuf)
        pltpu.sync_copy(ones, hist.at[idxbuf], add=True)
        return c
    lax.fori_loop(0, CHUNKS_16, body, 0)
    plsc.subcore_barrier()

    @pl.when(cid == 0)
    def _():
        tbase = sid * ROWS_PER_TILE
        pltpu.sync_copy(hist.at[pl.ds(tbase, ROWS_PER_TILE)],
                        hist_out.at[pl.ds(tbase, ROWS_PER_TILE)])


def _mm_body(x_ref, w_ref, h_ref, o_ref, d_ref):
    d = lax.rsqrt(h_ref[...] + 1.0)
    d_ref[...] = d
    s = jnp.dot(x_ref[...], w_ref[...], preferred_element_type=jnp.float32)
    o_ref[...] = s * d


@functools.partial(
    pl.kernel,
    out_type=jax.ShapeDtypeStruct((NC * ACC_N, F), jnp.float32),
    mesh=_MESH,
    scratch_types=[
        pltpu.VMEM((CH,), jnp.int32),
        pltpu.VMEM((CH,), jnp.int32),
        pltpu.VMEM((CH, F), jnp.float32),
        pltpu.VMEM((64, F), jnp.float32),
        pltpu.VMEM_SHARED((ACC_N, F), jnp.float32),
        pltpu.SemaphoreType.DMA,
    ],
)
def _edge_scatter(scaled_hbm, row_hbm, col_hbm, acc_out, colbuf, rowbuf, rows,
                  zbuf, acc, sem):
    cid = lax.axis_index("c")
    sid = lax.axis_index("s")
    wid = sid * NC + cid

    def zrow(i, c):
        for j in range(F // 16):
            zbuf[i, pl.ds(j * 16, 16)] = jnp.zeros((16,), jnp.float32)
        return c
    lax.fori_loop(0, 64, zrow, 0)

    def zcopy(k, c):
        pltpu.sync_copy(zbuf, acc.at[pl.ds(sid * ROWS_PER_TILE + k * 64, 64)])
        return c
    lax.fori_loop(0, ROWS_PER_TILE // 64, zcopy, 0)
    plsc.subcore_barrier()

    ebase = wid * (CHUNKS_32 * CH)

    def body(j, c):
        base = ebase + j * CH
        pltpu.sync_copy(col_hbm.at[pl.ds(base, CH)], colbuf)
        pltpu.sync_copy(row_hbm.at[pl.ds(base, CH)], rowbuf)
        pltpu.async_copy(scaled_hbm.at[colbuf], rows, sem).wait()
        pltpu.sync_copy(rows, acc.at[rowbuf], add=True)
        return c
    lax.fori_loop(0, CHUNKS_32, body, 0)
    plsc.subcore_barrier()

    tbase = sid * ROWS_PER_TILE
    pltpu.sync_copy(acc.at[pl.ds(tbase, ROWS_PER_TILE)],
                    acc_out.at[pl.ds(cid * ACC_N + tbase, ROWS_PER_TILE)])


def _combine_body(a_ref, s_ref, d_ref, b_ref, o_ref):
    acc = a_ref[0:N, :] + a_ref[ACC_N:ACC_N + N, :]
    o_ref[...] = d_ref[...] * (acc + s_ref[...]) + b_ref[...]


def kernel(x, edge_index, weight, bias):
    row = edge_index[0]
    col = edge_index[1]
    npad = E_PAD - E
    row_p = jnp.concatenate([row, jnp.full((npad,), N, jnp.int32)])
    col_p = jnp.concatenate([col, jnp.zeros((npad,), jnp.int32)])

    hist = _deg_hist(row_p)
    hist_col = hist[:N].reshape(N, 1)

    scaled, dinv_col = pl.pallas_call(
        _mm_body,
        out_shape=(jax.ShapeDtypeStruct((N, F), jnp.float32),
                   jax.ShapeDtypeStruct((N, 1), jnp.float32)),
    )(x, weight, hist_col)

    accs = _edge_scatter(scaled, row_p, col_p)

    out = pl.pallas_call(
        _combine_body,
        out_shape=jax.ShapeDtypeStruct((N, F), jnp.float32),
    )(accs, scaled, dinv_col, bias.reshape(1, F))
    return out


# speedup vs baseline: 33.7077x; 2.4621x over previous
import functools

import jax
import jax.numpy as jnp
from jax import lax
from jax.experimental import pallas as pl
from jax.experimental.pallas import tpu as pltpu
from jax.experimental.pallas import tpu_sc as plsc

N = 10000
F = 128
E = 320000

NC = 2
NS = 16
CH = 128
SL = 2

SEG = 16
CHUNKS = 80
E_PAD = NC * NS * CH * CHUNKS
SLABS = CHUNKS // SL
SSEG = SEG // SL
NSEG = CHUNKS // SEG

ROWS_PER_TILE = 640
ACC_N = NS * ROWS_PER_TILE

_MESH = plsc.VectorSubcoreMesh(core_axis_name="c", subcore_axis_name="s")


@functools.partial(
    pl.kernel,
    out_type=jax.ShapeDtypeStruct((NC * ACC_N,), jnp.float32),
    mesh=_MESH,
    scratch_types=[
        pltpu.VMEM((CHUNKS, CH), jnp.int32),
        pltpu.VMEM((CH,), jnp.float32),
        pltpu.VMEM((ROWS_PER_TILE,), jnp.float32),
        pltpu.VMEM_SHARED((ACC_N,), jnp.float32),
        pltpu.SemaphoreType.DMA,
    ],
)
def _deg_hist(row2d_hbm, hist_out, idx2d, ones, dbuf, hist, sem):
    cid = lax.axis_index("c")
    sid = lax.axis_index("s")

    def init(i, c):
        ones[pl.ds(i * 16, 16)] = jnp.ones((16,), jnp.float32)
        return c
    lax.fori_loop(0, CH // 16, init, 0)

    def zero(i, c):
        dbuf[pl.ds(i * 16, 16)] = jnp.zeros((16,), jnp.float32)
        return c
    lax.fori_loop(0, ROWS_PER_TILE // 16, zero, 0)
    pltpu.sync_copy(dbuf, hist.at[pl.ds(sid * ROWS_PER_TILE, ROWS_PER_TILE)])
    pltpu.sync_copy(row2d_hbm.at[pl.ds((cid * NS + sid) * CHUNKS, CHUNKS)], idx2d)
    plsc.subcore_barrier()

    W = 8

    def body(j, c):
        pltpu.async_copy(ones, hist.at[idx2d.at[j]], sem, add=True)

        @pl.when(j >= W)
        def _():
            pltpu.make_async_copy(ones, hist.at[idx2d.at[j - W]], sem).wait()
        return c
    lax.fori_loop(0, CHUNKS, body, 0)

    def drain(j, c):
        pltpu.make_async_copy(ones, hist.at[idx2d.at[j]], sem).wait()
        return c
    lax.fori_loop(CHUNKS - W, CHUNKS, drain, 0)
    plsc.subcore_barrier()

    tbase = sid * ROWS_PER_TILE
    pltpu.sync_copy(hist.at[pl.ds(tbase, ROWS_PER_TILE)],
                    hist_out.at[pl.ds(cid * ACC_N + tbase, ROWS_PER_TILE)])


def _mm_body(x_ref, w_ref, h0_ref, h1_ref, o_ref, d_ref):
    d = lax.rsqrt(h0_ref[...] + h1_ref[...] + 1.0)
    d_ref[...] = d
    s = jnp.dot(x_ref[...], w_ref[...], preferred_element_type=jnp.float32)
    o_ref[...] = s * d


@functools.partial(
    pl.kernel,
    out_type=jax.ShapeDtypeStruct((NC * ACC_N, F), jnp.float32),
    mesh=_MESH,
    scratch_types=[
        pltpu.VMEM((2 * SSEG * SL * CH,), jnp.int32),
        pltpu.VMEM((2 * SEG, CH), jnp.int32),
        pltpu.VMEM((SL * CH, F), jnp.float32),
        pltpu.VMEM_SHARED((ACC_N, F), jnp.float32),
        pltpu.SemaphoreType.DMA,
    ],
)
def _edge_scatter(scaled_hbm, row2d_hbm, col1d_hbm, acc_out, colidx, rowidx,
                  bigrows, acc, sem_i):
    cid = lax.axis_index("c")
    sid = lax.axis_index("s")
    wid = sid * NC + cid
    cbase_c = wid * SLABS * SL * CH
    cbase_r = wid * CHUNKS
    ESEG = SSEG * SL * CH

    def zrow(i, c):
        for j in range(F // 16):
            bigrows[i, pl.ds(j * 16, 16)] = jnp.zeros((16,), jnp.float32)
        return c
    lax.fori_loop(0, CH, zrow, 0)

    def zcopy(k, c):
        pltpu.sync_copy(bigrows.at[pl.ds(0, CH)],
                        acc.at[pl.ds(sid * ROWS_PER_TILE + k * CH, CH)])
        return c
    lax.fori_loop(0, ROWS_PER_TILE // CH, zcopy, 0)

    pltpu.sync_copy(col1d_hbm.at[pl.ds(cbase_c, ESEG)], colidx.at[pl.ds(0, ESEG)])
    pltpu.sync_copy(row2d_hbm.at[pl.ds(cbase_r, SEG)], rowidx.at[pl.ds(0, SEG)])

    def idx_prefetch(seg):
        offc = lax.rem(seg, 2) * ESEG
        offr = lax.rem(seg, 2) * SEG
        pltpu.async_copy(col1d_hbm.at[pl.ds(cbase_c + seg * ESEG, ESEG)],
                         colidx.at[pl.ds(offc, ESEG)], sem_i)
        pltpu.async_copy(row2d_hbm.at[pl.ds(cbase_r + seg * SEG, SEG)],
                         rowidx.at[pl.ds(offr, SEG)], sem_i)

    def idx_wait(seg):
        offc = lax.rem(seg, 2) * ESEG
        offr = lax.rem(seg, 2) * SEG
        pltpu.make_async_copy(col1d_hbm.at[pl.ds(cbase_c + seg * ESEG, ESEG)],
                              colidx.at[pl.ds(offc, ESEG)], sem_i).wait()
        pltpu.make_async_copy(row2d_hbm.at[pl.ds(cbase_r + seg * SEG, SEG)],
                              rowidx.at[pl.ds(offr, SEG)], sem_i).wait()

    idx_prefetch(1)
    plsc.subcore_barrier()

    def body(s, c):
        seg = s // SSEG
        cslot = lax.rem(s, 2 * SSEG) * (SL * CH)
        rslot = lax.rem(s * SL, 2 * SEG)
        crossing = (lax.rem(s, SSEG) == 0) & (s > 0)

        @pl.when(crossing)
        def _():
            idx_wait(seg)

        @pl.when(crossing & (seg + 1 <= NSEG - 1))
        def _():
            idx_prefetch(seg + 1)
        pltpu.sync_copy(scaled_hbm.at[colidx.at[pl.ds(cslot, SL * CH)]], bigrows)
        for j in range(SL):
            pltpu.sync_copy(bigrows.at[pl.ds(j * CH, CH)],
                            acc.at[rowidx.at[rslot + j]], add=True)
        return c
    lax.fori_loop(0, SLABS, body, 0)
    plsc.subcore_barrier()

    tbase = sid * ROWS_PER_TILE
    pltpu.sync_copy(acc.at[pl.ds(tbase, ROWS_PER_TILE)],
                    acc_out.at[pl.ds(cid * ACC_N + tbase, ROWS_PER_TILE)])


def _combine_body(a_ref, s_ref, d_ref, b_ref, o_ref):
    acc = a_ref[0:N, :] + a_ref[ACC_N:ACC_N + N, :]
    o_ref[...] = d_ref[...] * (acc + s_ref[...]) + b_ref[...]


def kernel(x, edge_index, weight, bias):
    row = edge_index[0]
    col = edge_index[1]
    npad = E_PAD - E
    seq = jnp.arange(npad, dtype=jnp.int32)
    pad_rows = N + seq % (ACC_N - N)
    pad_cols = seq % N
    row_p = jnp.concatenate([row, pad_rows])
    col_p = jnp.concatenate([col, pad_cols])
    row2d = row_p.reshape(-1, CH)

    hist = _deg_hist(row2d)
    h0_col = hist[:N].reshape(N, 1)
    h1_col = hist[ACC_N:ACC_N + N].reshape(N, 1)

    scaled, dinv_col = pl.pallas_call(
        _mm_body,
        out_shape=(jax.ShapeDtypeStruct((N, F), jnp.float32),
                   jax.ShapeDtypeStruct((N, 1), jnp.float32)),
    )(x, weight, h0_col, h1_col)

    accs = _edge_scatter(scaled, row2d, col_p)

    out = pl.pallas_call(
        _combine_body,
        out_shape=jax.ShapeDtypeStruct((N, F), jnp.float32),
    )(accs, scaled, dinv_col, bias.reshape(1, F))
    return out


# speedup vs baseline: 34.9995x; 1.0383x over previous
import functools

import jax
import jax.numpy as jnp
from jax import lax
from jax.experimental import pallas as pl
from jax.experimental.pallas import tpu as pltpu
from jax.experimental.pallas import tpu_sc as plsc

N = 10000
F = 128
E = 320000

NC = 2
NS = 16
CH = 128
NB = 2

SEG = 16
CHUNKS = 80
E_PAD = NC * NS * CH * CHUNKS
ROUNDS = CHUNKS // NB
NSEG = CHUNKS // SEG

ROWS_PER_TILE = 640
ACC_N = NS * ROWS_PER_TILE

_MESH = plsc.VectorSubcoreMesh(core_axis_name="c", subcore_axis_name="s")


@functools.partial(
    pl.kernel,
    out_type=jax.ShapeDtypeStruct((NC * ACC_N,), jnp.float32),
    mesh=_MESH,
    scratch_types=[
        pltpu.VMEM((CHUNKS, CH), jnp.int32),
        pltpu.VMEM((CH,), jnp.float32),
        pltpu.VMEM((ROWS_PER_TILE,), jnp.float32),
        pltpu.VMEM_SHARED((ACC_N,), jnp.float32),
        pltpu.SemaphoreType.DMA,
    ],
)
def _deg_hist(row2d_hbm, hist_out, idx2d, ones, dbuf, hist, sem):
    cid = lax.axis_index("c")
    sid = lax.axis_index("s")

    def init(i, c):
        ones[pl.ds(i * 16, 16)] = jnp.ones((16,), jnp.float32)
        return c
    lax.fori_loop(0, CH // 16, init, 0)

    def zero(i, c):
        dbuf[pl.ds(i * 16, 16)] = jnp.zeros((16,), jnp.float32)
        return c
    lax.fori_loop(0, ROWS_PER_TILE // 16, zero, 0)
    pltpu.sync_copy(dbuf, hist.at[pl.ds(sid * ROWS_PER_TILE, ROWS_PER_TILE)])
    pltpu.sync_copy(row2d_hbm.at[pl.ds((cid * NS + sid) * CHUNKS, CHUNKS)], idx2d)
    plsc.subcore_barrier()

    W = 8

    def body(j, c):
        pltpu.async_copy(ones, hist.at[idx2d.at[j]], sem, add=True)

        @pl.when(j >= W)
        def _():
            pltpu.make_async_copy(ones, hist.at[idx2d.at[j - W]], sem).wait()
        return c
    lax.fori_loop(0, CHUNKS, body, 0)

    def drain(j, c):
        pltpu.make_async_copy(ones, hist.at[idx2d.at[j]], sem).wait()
        return c
    lax.fori_loop(CHUNKS - W, CHUNKS, drain, 0)
    plsc.subcore_barrier()

    tbase = sid * ROWS_PER_TILE
    pltpu.sync_copy(hist.at[pl.ds(tbase, ROWS_PER_TILE)],
                    hist_out.at[pl.ds(cid * ACC_N + tbase, ROWS_PER_TILE)])


def _mm_body(x_ref, w_ref, h0_ref, h1_ref, o_ref, d_ref):
    d = lax.rsqrt(h0_ref[...] + h1_ref[...] + 1.0)
    d_ref[...] = d
    s = jnp.dot(x_ref[...], w_ref[...], preferred_element_type=jnp.float32)
    o_ref[...] = s * d


@functools.partial(
    pl.kernel,
    out_type=jax.ShapeDtypeStruct((NC * ACC_N, F), jnp.float32),
    mesh=_MESH,
    scratch_types=[
        pltpu.VMEM((2 * SEG, CH), jnp.int32),
        pltpu.VMEM((2 * SEG, CH), jnp.int32),
        pltpu.VMEM((CH, F), jnp.float32),
        pltpu.VMEM((CH, F), jnp.float32),
        pltpu.VMEM_SHARED((ACC_N, F), jnp.float32),
        pltpu.SemaphoreType.DMA,
        pltpu.SemaphoreType.DMA,
        pltpu.SemaphoreType.DMA,
        pltpu.SemaphoreType.DMA,
        pltpu.SemaphoreType.DMA,
    ],
)
def _edge_scatter(scaled_hbm, row2d_hbm, col2d_hbm, acc_out, colidx, rowidx,
                  rows0, rows1, acc,
                  sg0, sg1, ss0, ss1, sem_i):
    cid = lax.axis_index("c")
    sid = lax.axis_index("s")
    wid = sid * NC + cid
    cbase = wid * CHUNKS
    rows = (rows0, rows1)
    sg = (sg0, sg1)
    ss = (ss0, ss1)

    def zrow(i, c):
        for j in range(F // 16):
            rows0[i, pl.ds(j * 16, 16)] = jnp.zeros((16,), jnp.float32)
        return c
    lax.fori_loop(0, CH, zrow, 0)

    def zcopy(k, c):
        pltpu.sync_copy(rows0, acc.at[pl.ds(sid * ROWS_PER_TILE + k * CH, CH)])
        return c
    nfull = ROWS_PER_TILE // CH
    lax.fori_loop(0, nfull, zcopy, 0)
    rem_rows = ROWS_PER_TILE - nfull * CH
    if rem_rows:
        pltpu.sync_copy(
            rows0.at[pl.ds(0, rem_rows)],
            acc.at[pl.ds(sid * ROWS_PER_TILE + nfull * CH, rem_rows)])

    pltpu.sync_copy(col2d_hbm.at[pl.ds(cbase, SEG)], colidx.at[pl.ds(0, SEG)])
    pltpu.sync_copy(row2d_hbm.at[pl.ds(cbase, SEG)], rowidx.at[pl.ds(0, SEG)])

    def idx_prefetch(seg):
        off = lax.rem(seg, 2) * SEG
        pltpu.async_copy(col2d_hbm.at[pl.ds(cbase + seg * SEG, SEG)],
                         colidx.at[pl.ds(off, SEG)], sem_i)
        pltpu.async_copy(row2d_hbm.at[pl.ds(cbase + seg * SEG, SEG)],
                         rowidx.at[pl.ds(off, SEG)], sem_i)

    def idx_wait(seg):
        off = lax.rem(seg, 2) * SEG
        pltpu.make_async_copy(col2d_hbm.at[pl.ds(cbase + seg * SEG, SEG)],
                              colidx.at[pl.ds(off, SEG)], sem_i).wait()
        pltpu.make_async_copy(row2d_hbm.at[pl.ds(cbase + seg * SEG, SEG)],
                              rowidx.at[pl.ds(off, SEG)], sem_i).wait()

    idx_prefetch(1)
    plsc.subcore_barrier()

    def gather(b, chunk):
        slot = lax.rem(chunk, 2 * SEG)
        pltpu.async_copy(scaled_hbm.at[colidx.at[slot]], rows[b], sg[b])

    def gather_wait(b, chunk):
        slot = lax.rem(chunk, 2 * SEG)
        pltpu.make_async_copy(scaled_hbm.at[colidx.at[slot]], rows[b],
                              sg[b]).wait()

    def scatter(b, chunk):
        slot = lax.rem(chunk, 2 * SEG)
        pltpu.async_copy(rows[b], acc.at[rowidx.at[slot]], ss[b], add=True)

    def scatter_wait(b, chunk):
        slot = lax.rem(chunk, 2 * SEG)
        pltpu.make_async_copy(rows[b], acc.at[rowidx.at[slot]], ss[b]).wait()

    for b in range(NB):
        gather(b, b)

    def body(r, c):
        for b in range(NB):
            gather_wait(b, r * NB + b)
            scatter(b, r * NB + b)
        seg_next = (r + 1) * NB // SEG
        crossing = lax.rem(r, SEG // NB) == SEG // NB - 1

        @pl.when(crossing)
        def _():
            idx_wait(seg_next)
        for b in range(NB):
            scatter_wait(b, r * NB + b)
            gather(b, (r + 1) * NB + b)

        @pl.when(crossing & (seg_next + 1 <= NSEG - 1))
        def _():
            idx_prefetch(seg_next + 1)
        return c
    lax.fori_loop(0, ROUNDS - 1, body, 0)

    for b in range(NB):
        gather_wait(b, (ROUNDS - 1) * NB + b)
        scatter(b, (ROUNDS - 1) * NB + b)
    for b in range(NB):
        scatter_wait(b, (ROUNDS - 1) * NB + b)
    plsc.subcore_barrier()

    tbase = sid * ROWS_PER_TILE
    pltpu.sync_copy(acc.at[pl.ds(tbase, ROWS_PER_TILE)],
                    acc_out.at[pl.ds(cid * ACC_N + tbase, ROWS_PER_TILE)])


def _combine_body(a_ref, s_ref, d_ref, b_ref, o_ref):
    acc = a_ref[0:N, :] + a_ref[ACC_N:ACC_N + N, :]
    o_ref[...] = d_ref[...] * (acc + s_ref[...]) + b_ref[...]


def kernel(x, edge_index, weight, bias):
    row = edge_index[0]
    col = edge_index[1]
    npad = E_PAD - E
    seq = jnp.arange(npad, dtype=jnp.int32)
    pad_rows = N + seq % (ACC_N - N)
    pad_cols = seq % N
    row2d = jnp.concatenate([row, pad_rows]).reshape(-1, CH)
    col2d = jnp.concatenate([col, pad_cols]).reshape(-1, CH)

    hist = _deg_hist(row2d)
    h0_col = hist[:N].reshape(N, 1)
    h1_col = hist[ACC_N:ACC_N + N].reshape(N, 1)

    scaled, dinv_col = pl.pallas_call(
        _mm_body,
        out_shape=(jax.ShapeDtypeStruct((N, F), jnp.float32),
                   jax.ShapeDtypeStruct((N, 1), jnp.float32)),
    )(x, weight, h0_col, h1_col)

    accs = _edge_scatter(scaled, row2d, col2d)

    out = pl.pallas_call(
        _combine_body,
        out_shape=jax.ShapeDtypeStruct((N, F), jnp.float32),
    )(accs, scaled, dinv_col, bias.reshape(1, F))
    return out
